# Initial kernel scaffold; baseline (speedup 1.0000x reference)
import jax
import jax.numpy as jnp
from jax.experimental import pallas as pl


def kernel(target_node_ids, all_node_features, neighbor_ids_l1, neighbor_ids_l2, W1, b1, W2, b2):
    raise NotImplementedError("write your pallas kernel here")


# speedup vs baseline: 1.9437x; 1.9437x over previous
import functools

import jax
import jax.numpy as jnp
from jax import lax
from jax.experimental import pallas as pl
from jax.experimental.pallas import tpu as pltpu
from jax.experimental.pallas import tpu_sc as plsc

N_NODES = 100000
D = 128
B = 16384
F1 = 10
F2 = 5
NV = D // 16


def _reduce_rows(rows_v, out_v, n_elems, fanout):
    def elem_body(e, carry):
        base = e * fanout
        accs = tuple(rows_v[base, pl.ds(v * 16, 16)] for v in range(NV))

        def j_body(j, accs):
            return tuple(
                accs[v] + rows_v[base + j, pl.ds(v * 16, 16)] for v in range(NV)
            )

        accs = lax.fori_loop(1, fanout, j_body, accs)
        for v in range(NV):
            out_v[e, pl.ds(v * 16, 16)] = accs[v]
        return carry

    lax.fori_loop(0, n_elems, elem_body, 0)


def _sc_gather_aggregate(tids, feats, n1_flat, n2_flat):
    info = plsc.get_sparse_core_info()
    nw = info.num_cores * info.num_subcores
    per_w = B // nw

    TCH = 128
    E1 = 8
    E2 = 16
    R1 = E1 * F1
    R2 = E2 * F2

    mesh = plsc.VectorSubcoreMesh(core_axis_name="c", subcore_axis_name="s")

    @functools.partial(
        pl.kernel,
        out_type=[
            jax.ShapeDtypeStruct((B, D), jnp.float32),
            jax.ShapeDtypeStruct((B, D), jnp.float32),
            jax.ShapeDtypeStruct((B, D), jnp.float32),
        ],
        mesh=mesh,
        scratch_types=[
            pltpu.VMEM((TCH,), jnp.int32),
            pltpu.VMEM((TCH, D), jnp.float32),
            pltpu.VMEM((R1,), jnp.int32),
            pltpu.VMEM((R1, D), jnp.float32),
            pltpu.VMEM((E1, D), jnp.float32),
            pltpu.VMEM((R2,), jnp.int32),
            pltpu.VMEM((R2, D), jnp.float32),
            pltpu.VMEM((E2, D), jnp.float32),
            pltpu.SemaphoreType.DMA,
        ],
    )
    def sc_kernel(
        tids_hbm, feats_hbm, n1_hbm, n2_hbm,
        ht_hbm, s1_hbm, s2_hbm,
        idxT, rowsT, idx1, rows1, out1, idx2, rows2, out2, sem,
    ):
        wid = lax.axis_index("s") * info.num_cores + lax.axis_index("c")
        wbase = wid * per_w

        def t_body(c, carry):
            base = wbase + c * TCH
            pltpu.sync_copy(tids_hbm.at[pl.ds(base, TCH)], idxT)
            pltpu.async_copy(feats_hbm.at[idxT], rowsT, sem).wait()
            pltpu.sync_copy(rowsT, ht_hbm.at[pl.ds(base, TCH)])
            return carry

        lax.fori_loop(0, per_w // TCH, t_body, 0)

        def l1_body(r, carry):
            ebase = wbase + r * E1
            pltpu.sync_copy(n1_hbm.at[pl.ds(ebase * F1, R1)], idx1)
            pltpu.async_copy(feats_hbm.at[idx1], rows1, sem).wait()
            _reduce_rows(rows1, out1, E1, F1)
            pltpu.sync_copy(out1, s1_hbm.at[pl.ds(ebase, E1)])
            return carry

        lax.fori_loop(0, per_w // E1, l1_body, 0)

        def l2_body(r, carry):
            ebase = wbase + r * E2
            pltpu.sync_copy(n2_hbm.at[pl.ds(ebase * F2, R2)], idx2)
            pltpu.async_copy(feats_hbm.at[idx2], rows2, sem).wait()
            _reduce_rows(rows2, out2, E2, F2)
            pltpu.sync_copy(out2, s2_hbm.at[pl.ds(ebase, E2)])
            return carry

        lax.fori_loop(0, per_w // E2, l2_body, 0)

    return sc_kernel(tids, feats, n1_flat, n2_flat)


def _tc_dense(ht, s1, s2, W1a, W1b, b1, W2a, W2b, b2):
    BLK = 1024
    grid = (B // BLK,)

    def body(ht_r, s1_r, s2_r, w1a_r, w1b_r, b1_r, w2a_r, w2b_r, b2_r, out_r):
        h = ht_r[...]
        a1 = s1_r[...] / 10.0
        x1 = (
            jnp.dot(h, w1a_r[...], preferred_element_type=jnp.float32)
            + jnp.dot(a1, w1b_r[...], preferred_element_type=jnp.float32)
            + b1_r[...]
        )
        h1 = jnp.maximum(x1, 0.0)
        a2 = s2_r[...] / 5.0
        x2 = (
            jnp.dot(h1, w2a_r[...], preferred_element_type=jnp.float32)
            + jnp.dot(a2, w2b_r[...], preferred_element_type=jnp.float32)
            + b2_r[...]
        )
        out_r[...] = jnp.maximum(x2, 0.0)

    row_spec = pl.BlockSpec((BLK, D), lambda i: (i, 0))
    full = lambda shape: pl.BlockSpec(shape, lambda i: tuple(0 for _ in shape))
    return pl.pallas_call(
        body,
        grid=grid,
        in_specs=[
            row_spec, row_spec, row_spec,
            full((D, 64)), full((D, 64)), full((1, 64)),
            full((64, 64)), full((D, 64)), full((1, 64)),
        ],
        out_specs=pl.BlockSpec((BLK, 64), lambda i: (i, 0)),
        out_shape=jax.ShapeDtypeStruct((B, 64), jnp.float32),
    )(ht, s1, s2, W1a, W1b, b1, W2a, W2b, b2)


def kernel(target_node_ids, all_node_features, neighbor_ids_l1, neighbor_ids_l2,
           W1, b1, W2, b2):
    tids = target_node_ids.astype(jnp.int32)
    n1_flat = neighbor_ids_l1.astype(jnp.int32).reshape(-1)
    n2_flat = neighbor_ids_l2.astype(jnp.int32).reshape(-1)
    ht, s1, s2 = _sc_gather_aggregate(tids, all_node_features, n1_flat, n2_flat)
    return _tc_dense(
        ht, s1, s2,
        W1[:D], W1[D:], b1.reshape(1, -1),
        W2[:64], W2[64:], b2.reshape(1, -1),
    )


# speedup vs baseline: 3.4321x; 1.7658x over previous
import functools

import jax
import jax.numpy as jnp
from jax import lax
from jax.experimental import pallas as pl
from jax.experimental.pallas import tpu as pltpu
from jax.experimental.pallas import tpu_sc as plsc

N_NODES = 100000
D = 128
B = 16384
F1 = 10
F2 = 5
NV = D // 16

PER_W = B // 32
TCH = 128
RT = PER_W // TCH
E1 = 8
R1 = PER_W // E1
E2 = 16
R2 = PER_W // E2


def _reduce_rows(rows_v, out_v, n_elems, fanout):
    def elem_body(e, carry):
        base = e * fanout
        accs = [rows_v[base, pl.ds(v * 16, 16)] for v in range(NV)]
        for j in range(1, fanout):
            for v in range(NV):
                accs[v] = accs[v] + rows_v[base + j, pl.ds(v * 16, 16)]
        for v in range(NV):
            out_v[e, pl.ds(v * 16, 16)] = accs[v]
        return carry

    lax.fori_loop(0, n_elems, elem_body, 0)


def _sc_gather_aggregate(tids2d, feats, n1_2d, n2_2d):
    info = plsc.get_sparse_core_info()
    nw = info.num_cores * info.num_subcores
    assert nw == 32

    mesh = plsc.VectorSubcoreMesh(core_axis_name="c", subcore_axis_name="s")

    @functools.partial(
        pl.kernel,
        out_type=[
            jax.ShapeDtypeStruct((B, D), jnp.float32),
            jax.ShapeDtypeStruct((B, D), jnp.float32),
            jax.ShapeDtypeStruct((B, D), jnp.float32),
        ],
        mesh=mesh,
        scratch_types=[
            pltpu.VMEM((RT, TCH), jnp.int32),
            pltpu.VMEM((TCH, D), jnp.float32),
            pltpu.VMEM((TCH, D), jnp.float32),
            pltpu.VMEM((R1, E1 * F1), jnp.int32),
            pltpu.VMEM((E1 * F1, D), jnp.float32),
            pltpu.VMEM((E1 * F1, D), jnp.float32),
            pltpu.VMEM((E1, D), jnp.float32),
            pltpu.VMEM((E1, D), jnp.float32),
            pltpu.VMEM((R2, E2 * F2), jnp.int32),
            pltpu.VMEM((E2 * F2, D), jnp.float32),
            pltpu.VMEM((E2 * F2, D), jnp.float32),
            pltpu.VMEM((E2, D), jnp.float32),
            pltpu.VMEM((E2, D), jnp.float32),
            pltpu.SemaphoreType.DMA,
            pltpu.SemaphoreType.DMA,
            pltpu.SemaphoreType.DMA,
            pltpu.SemaphoreType.DMA,
        ],
    )
    def sc_kernel(
        tids_hbm, feats_hbm, n1_hbm, n2_hbm,
        ht_hbm, s1_hbm, s2_hbm,
        idxT, rowsT0, rowsT1, idx1, rows1a, rows1b, out1a, out1b,
        idx2, rows2a, rows2b, out2a, out2b,
        sg0, sg1, so0, so1,
    ):
        wid = lax.axis_index("s") * info.num_cores + lax.axis_index("c")
        wbase = wid * PER_W
        sg = (sg0, sg1)
        so = (so0, so1)

        pltpu.sync_copy(tids_hbm.at[pl.ds(wid * RT, RT)], idxT)
        rowsT = (rowsT0, rowsT1)
        for half in range(2):
            pltpu.async_copy(feats_hbm.at[idxT.at[half]], rowsT[half], sg[half])

        def t_body(rr, carry):
            for half in range(2):
                r = 2 * rr + half
                buf = rowsT[half]
                pltpu.make_async_copy(
                    feats_hbm.at[idxT.at[r]], buf, sg[half]
                ).wait()
                base = wbase + r * TCH
                pltpu.async_copy(buf, ht_hbm.at[pl.ds(base, TCH)], so[half])

                @pl.when(r + 2 < RT)
                def _():
                    pltpu.make_async_copy(
                        buf, ht_hbm.at[pl.ds(base, TCH)], so[half]
                    ).wait()
                    pltpu.async_copy(
                        feats_hbm.at[idxT.at[r + 2]], buf, sg[half]
                    )

            return carry

        lax.fori_loop(0, RT // 2, t_body, 0)
        for half in range(2):
            r = RT - 2 + half
            pltpu.make_async_copy(
                rowsT[half], ht_hbm.at[pl.ds(wbase + r * TCH, TCH)], so[half]
            ).wait()

        def run_pass(idx_v, rows, outs, idx_hbm, out_hbm, rounds, elems, fanout):
            rows_per = elems * fanout
            pltpu.sync_copy(idx_hbm.at[pl.ds(wid * rounds, rounds)], idx_v)
            for half in range(2):
                pltpu.async_copy(
                    feats_hbm.at[idx_v.at[half]], rows[half], sg[half]
                )

            def body(rr, carry):
                for half in range(2):
                    r = 2 * rr + half
                    buf = rows[half]
                    outb = outs[half]
                    pltpu.make_async_copy(
                        feats_hbm.at[idx_v.at[r]], buf, sg[half]
                    ).wait()
                    ebase = wbase + r * elems

                    @pl.when(rr >= 1)
                    def _():
                        pltpu.make_async_copy(
                            outb, out_hbm.at[pl.ds(ebase, elems)], so[half]
                        ).wait()

                    _reduce_rows(buf, outb, elems, fanout)

                    @pl.when(r + 2 < rounds)
                    def _():
                        pltpu.async_copy(
                            feats_hbm.at[idx_v.at[r + 2]], buf, sg[half]
                        )

                    pltpu.async_copy(
                        outb, out_hbm.at[pl.ds(ebase, elems)], so[half]
                    )
                return carry

            lax.fori_loop(0, rounds // 2, body, 0)
            for half in range(2):
                r = rounds - 2 + half
                pltpu.make_async_copy(
                    outs[half],
                    out_hbm.at[pl.ds(wbase + r * elems, elems)],
                    so[half],
                ).wait()

        run_pass(idx1, (rows1a, rows1b), (out1a, out1b),
                 n1_hbm, s1_hbm, R1, E1, F1)
        run_pass(idx2, (rows2a, rows2b), (out2a, out2b),
                 n2_hbm, s2_hbm, R2, E2, F2)

    return sc_kernel(tids2d, feats, n1_2d, n2_2d)


def _tc_dense(ht, s1, s2, W1a, W1b, b1, W2a, W2b, b2):
    BLK = 1024
    grid = (B // BLK,)

    def body(ht_r, s1_r, s2_r, w1a_r, w1b_r, b1_r, w2a_r, w2b_r, b2_r, out_r):
        h = ht_r[...]
        a1 = s1_r[...] / 10.0
        x1 = (
            jnp.dot(h, w1a_r[...], preferred_element_type=jnp.float32)
            + jnp.dot(a1, w1b_r[...], preferred_element_type=jnp.float32)
            + b1_r[...]
        )
        h1 = jnp.maximum(x1, 0.0)
        a2 = s2_r[...] / 5.0
        x2 = (
            jnp.dot(h1, w2a_r[...], preferred_element_type=jnp.float32)
            + jnp.dot(a2, w2b_r[...], preferred_element_type=jnp.float32)
            + b2_r[...]
        )
        out_r[...] = jnp.maximum(x2, 0.0)

    row_spec = pl.BlockSpec((BLK, D), lambda i: (i, 0))
    full = lambda shape: pl.BlockSpec(shape, lambda i: tuple(0 for _ in shape))
    return pl.pallas_call(
        body,
        grid=grid,
        in_specs=[
            row_spec, row_spec, row_spec,
            full((D, 64)), full((D, 64)), full((1, 64)),
            full((64, 64)), full((D, 64)), full((1, 64)),
        ],
        out_specs=pl.BlockSpec((BLK, 64), lambda i: (i, 0)),
        out_shape=jax.ShapeDtypeStruct((B, 64), jnp.float32),
    )(ht, s1, s2, W1a, W1b, b1, W2a, W2b, b2)


def kernel(target_node_ids, all_node_features, neighbor_ids_l1, neighbor_ids_l2,
           W1, b1, W2, b2):
    tids2d = target_node_ids.astype(jnp.int32).reshape(B // TCH, TCH)
    n1_2d = neighbor_ids_l1.astype(jnp.int32).reshape(B // E1, E1 * F1)
    n2_2d = neighbor_ids_l2.astype(jnp.int32).reshape(B // E2, E2 * F2)
    ht, s1, s2 = _sc_gather_aggregate(tids2d, all_node_features, n1_2d, n2_2d)
    return _tc_dense(
        ht, s1, s2,
        W1[:D], W1[D:], b1.reshape(1, -1),
        W2[:64], W2[64:], b2.reshape(1, -1),
    )
